# baseline (speedup 1.0000x reference)
import jax
import jax.numpy as jnp
from jax.experimental import pallas as pl

N_NODES = 10000
N_CLUSTERS = 16


def _mlp(x, params, plain_last):
    n = len(params)
    for i, (W, b) in enumerate(params):
        x = x @ W + b
        if (not plain_last) or (i < n - 1):
            x = jax.nn.relu(x)
    return x


def _pointnet_conv(x, pos, edge_index, local_p, global_p):
    src = edge_index[0]
    dst = edge_index[1]
    m = jnp.concatenate(
        [jnp.take(x, src, axis=0), jnp.take(pos, src, axis=0) - jnp.take(pos, dst, axis=0)], axis=1)
    m = _mlp(m, local_p, plain_last=False)
    agg = jax.ops.segment_max(m, dst, num_segments=N_NODES)
    agg = jnp.where(jnp.isneginf(agg), 0.0, agg)
    return _mlp(agg, global_p, plain_last=False)


def _final_body(g_ref, w1_ref, b1_ref, w2_ref, b2_ref, o_ref):
    h = jnp.maximum(g_ref[...] @ w1_ref[...] + b1_ref[...], 0.0)
    o_ref[...] = h @ w2_ref[...] + b2_ref[...]


def kernel(x, pos, params, clusterID, edge_index):
    x1 = _pointnet_conv(x, pos, edge_index, params["sa1_local"], params["sa1_global"])
    x2 = _pointnet_conv(x1, pos, edge_index, params["sa2_local"], params["sa2_global"])
    h = _mlp(jnp.concatenate([x2, pos], axis=1), params["sa3"], plain_last=False)
    g = jax.ops.segment_max(h, clusterID, num_segments=N_CLUSTERS)
    g = jnp.where(jnp.isneginf(g), 0.0, g)
    (W1, b1), (W2, b2) = params["final"]
    out = pl.pallas_call(
        _final_body,
        out_shape=jax.ShapeDtypeStruct((N_CLUSTERS, W2.shape[1]), jnp.float32),
    )(g, W1, b1.reshape(1, -1), W2, b2.reshape(1, -1))
    return out


# speedup vs baseline: 1.3209x; 1.3209x over previous
import functools

import jax
import jax.numpy as jnp
from jax import lax
from jax.experimental import pallas as pl
from jax.experimental.pallas import tpu as pltpu
from jax.experimental.pallas import tpu_sc as plsc

N_NODES = 10000
N_EDGES = 320000
N_CLUSTERS = 16
NC, NS = 2, 16
NW = NC * NS
N_PAD = 10240
R_PER_W = N_PAD // NW


def _mesh():
    return plsc.VectorSubcoreMesh(
        core_axis_name="c", subcore_axis_name="s", num_cores=NC, num_subcores=NS)


def _wid():
    return lax.axis_index("s") * NC + lax.axis_index("c")


def _sc_gather(table, idx, *, C, tc_tiling=True):
    E = idx.shape[0]
    V, D = table.shape
    Ew = E // NW
    nchunks = Ew // C

    @functools.partial(
        pl.kernel,
        mesh=_mesh(),
        compiler_params=pltpu.CompilerParams(use_tc_tiling_on_sc=tc_tiling),
        out_type=jax.ShapeDtypeStruct((E, D), jnp.float32),
        scratch_types=[
            pltpu.VMEM((Ew,), jnp.int32),
            pltpu.VMEM((C, D), jnp.float32),
            pltpu.SemaphoreType.DMA,
        ],
    )
    def k(table_hbm, idx_hbm, out_hbm, ibuf, rbuf, sem):
        base = _wid() * Ew
        pltpu.sync_copy(idx_hbm.at[pl.ds(base, Ew)], ibuf)

        def chunk(i, _):
            off = i * C
            pltpu.async_copy(table_hbm.at[ibuf.at[pl.ds(off, C)]], rbuf, sem).wait()
            pltpu.sync_copy(rbuf, out_hbm.at[pl.ds(base + off, C)])
            return 0

        lax.fori_loop(0, nchunks, chunk, 0)

    return k(table, idx)


def _sc_segmax(z, dst, *, K, CH):
    E, D = z.shape
    nchunks = E // CH
    ngroups = CH // 16
    nsl = D // 16

    @functools.partial(
        pl.kernel,
        mesh=_mesh(),
        compiler_params=pltpu.CompilerParams(needs_layout_passes=False),
        out_type=jax.ShapeDtypeStruct((N_PAD, D), jnp.float32),
        scratch_types=[
            pltpu.VMEM((R_PER_W, D), jnp.float32),
            pltpu.VMEM((CH,), jnp.int32),
            pltpu.VMEM((K + 16,), jnp.int32),
            pltpu.VMEM((K + 16,), jnp.int32),
            pltpu.VMEM((K, D), jnp.float32),
            pltpu.VMEM((32,), jnp.int32),
            pltpu.SemaphoreType.DMA,
        ],
    )
    def k(z_hbm, dst_hbm, out_hbm, tab, dbuf, eidb, ldb, rows, zbuf, sem):
        lo = _wid() * R_PER_W
        hi = lo + R_PER_W
        zero16 = jnp.zeros((16,), jnp.int32)
        one16 = jnp.ones((16,), jnp.int32)
        iota16 = lax.iota(jnp.int32, 16)
        zero16f = jnp.zeros((16,), jnp.float32)
        lo16 = jnp.full((16,), lo, jnp.int32)
        hi16 = jnp.full((16,), hi, jnp.int32)

        def init_tab(r, _):
            for s in range(nsl):
                tab[r, pl.ds(s * 16, 16)] = zero16f
            return 0

        lax.fori_loop(0, R_PER_W, init_tab, 0)

        for j in range(K // 16 + 1):
            eidb[pl.ds(j * 16, 16)] = iota16
        zbuf[pl.ds(0, 16)] = zero16

        def prefix(mi):
            p = mi
            for sh in (1, 2, 4, 8):
                zbuf[pl.ds(16, 16)] = p
                p = p + zbuf[pl.ds(16 - sh, 16)]
            return p

        def flush(n):
            pltpu.async_copy(z_hbm.at[eidb.at[pl.ds(0, K)]], rows, sem).wait()

            def per_edge(i, _):
                ld = ldb[pl.ds(i, 16)][0]
                for s in range(nsl):
                    sl = pl.ds(s * 16, 16)
                    tab[ld, sl] = jnp.maximum(tab[ld, sl], rows[i, sl])
                return 0

            lax.fori_loop(0, n, per_edge, 0)

        def chunk(ci, carry):
            pltpu.sync_copy(dst_hbm.at[pl.ds(ci * CH, CH)], dbuf)

            def group(g, carry):
                off, eidv = carry
                d = dbuf[pl.ds(g * 16, 16)]
                mi = (jnp.where(d >= lo16, one16, zero16)
                      * jnp.where(d < hi16, one16, zero16))
                cum = prefix(mi)
                matched = jnp.where(mi > zero16,
                                    cum + jnp.full((16,), off - 1, jnp.int32),
                                    jnp.full((16,), K, jnp.int32) + iota16)
                plsc.store_scatter(eidb, [matched], eidv)
                plsc.store_scatter(ldb, [matched], d - lo16)
                off = off + cum[15]
                eidv = eidv + jnp.full((16,), 16, jnp.int32)

                def do_flush():
                    flush(off)
                    return jnp.int32(0)

                off = lax.cond(off > K - 17, do_flush, lambda: off)
                return (off, eidv)

            return lax.fori_loop(0, ngroups, group, carry)

        off, _ = lax.fori_loop(0, nchunks, chunk, (jnp.int32(0), iota16))
        flush(off)
        pltpu.sync_copy(tab, out_hbm.at[pl.ds(lo, R_PER_W)])

    return k(z, dst)


def _node_block_spec(bn, d):
    return pl.BlockSpec((bn, d), lambda i: (i, 0))


def _full_spec(shape):
    return pl.BlockSpec(shape, lambda i: tuple(0 for _ in shape))


def _tc_prep1_body(x_ref, p_ref, wx_ref, wp_ref, b_ref, a_ref):
    a_ref[...] = (x_ref[...] @ wx_ref[...] + p_ref[...] @ wp_ref[...]
                  + b_ref[...])


def _tc_prep1(x_p, pos16, wx, wp16, b, bn):
    n = x_p.shape[0]
    dout = wx.shape[1]
    return pl.pallas_call(
        _tc_prep1_body,
        grid=(n // bn,),
        in_specs=[
            _node_block_spec(bn, x_p.shape[1]),
            _node_block_spec(bn, 16),
            _full_spec(wx.shape),
            _full_spec(wp16.shape),
            _full_spec((1, dout)),
        ],
        out_specs=_node_block_spec(bn, dout),
        out_shape=jax.ShapeDtypeStruct((n, dout), jnp.float32),
    )(x_p, pos16, wx, wp16, b.reshape(1, -1))


def _tc_edge_body(ma_ref, mp_ref, wp_ref, w2_ref, b2_ref, z_ref):
    m = jnp.maximum(ma_ref[...] - mp_ref[...] @ wp_ref[...], 0.0)
    z_ref[...] = m @ w2_ref[...] + b2_ref[...]


def _tc_edge(ma, mp, wp16, w2, b2, be):
    e, d = ma.shape
    dout = w2.shape[1]
    return pl.pallas_call(
        _tc_edge_body,
        grid=(e // be,),
        in_specs=[
            _node_block_spec(be, d),
            _node_block_spec(be, 16),
            _full_spec(wp16.shape),
            _full_spec(w2.shape),
            _full_spec((1, dout)),
        ],
        out_specs=_node_block_spec(be, dout),
        out_shape=jax.ShapeDtypeStruct((e, dout), jnp.float32),
    )(ma, mp, wp16, w2, b2.reshape(1, -1))


def _tc_prep2_body(agg_ref, p_ref, wg_ref, bg_ref, wx_ref, wp_ref, b_ref, a_ref):
    x1 = jnp.maximum(agg_ref[...] @ wg_ref[...] + bg_ref[...], 0.0)
    a_ref[...] = x1 @ wx_ref[...] + p_ref[...] @ wp_ref[...] + b_ref[...]


def _tc_prep2(agg, pos16, wg, bg, wx, wp16, b, bn):
    n = agg.shape[0]
    dout = wx.shape[1]
    return pl.pallas_call(
        _tc_prep2_body,
        grid=(n // bn,),
        in_specs=[
            _node_block_spec(bn, agg.shape[1]),
            _node_block_spec(bn, 16),
            _full_spec(wg.shape),
            _full_spec((1, wg.shape[1])),
            _full_spec(wx.shape),
            _full_spec(wp16.shape),
            _full_spec((1, dout)),
        ],
        out_specs=_node_block_spec(bn, dout),
        out_shape=jax.ShapeDtypeStruct((n, dout), jnp.float32),
    )(agg, pos16, wg, bg.reshape(1, -1), wx, wp16, b.reshape(1, -1))


def _tc_tail_body(agg_ref, p_ref, cid_ref, wg_ref, bg_ref, v1x_ref, v1p_ref,
                  c1_ref, v2_ref, c2_ref, g_ref):
    x2 = jnp.maximum(agg_ref[...] @ wg_ref[...] + bg_ref[...], 0.0)
    h1 = jnp.maximum(x2 @ v1x_ref[...] + p_ref[...] @ v1p_ref[...]
                     + c1_ref[...], 0.0)
    h = jnp.maximum(h1 @ v2_ref[...] + c2_ref[...], 0.0)

    @pl.when(pl.program_id(0) == 0)
    def _():
        g_ref[...] = jnp.zeros_like(g_ref)

    cid = cid_ref[...]
    for c in range(N_CLUSTERS):
        sel = jnp.where(cid == float(c), h, 0.0)
        g_ref[c, :] = jnp.maximum(g_ref[c, :], jnp.max(sel, axis=0))


def _tc_tail(agg, pos16, cidf, wg, bg, v1x, v1p16, c1, v2, c2, bn):
    n = agg.shape[0]
    dh = v2.shape[1]
    return pl.pallas_call(
        _tc_tail_body,
        grid=(n // bn,),
        in_specs=[
            _node_block_spec(bn, agg.shape[1]),
            _node_block_spec(bn, 16),
            _node_block_spec(bn, 1),
            _full_spec(wg.shape),
            _full_spec((1, wg.shape[1])),
            _full_spec(v1x.shape),
            _full_spec(v1p16.shape),
            _full_spec((1, dh)),
            _full_spec(v2.shape),
            _full_spec((1, dh)),
        ],
        out_specs=pl.BlockSpec((N_CLUSTERS, dh), lambda i: (0, 0)),
        out_shape=jax.ShapeDtypeStruct((N_CLUSTERS, dh), jnp.float32),
    )(agg, pos16, cidf, wg, bg.reshape(1, -1), v1x, v1p16, c1.reshape(1, -1),
      v2, c2.reshape(1, -1))


def _tc_final_body(g_ref, w1_ref, b1_ref, w2_ref, b2_ref, o_ref):
    h = jnp.maximum(g_ref[...] @ w1_ref[...] + b1_ref[...], 0.0)
    o_ref[...] = h @ w2_ref[...] + b2_ref[...]


def _tc_final(g, w1, b1, w2, b2):
    return pl.pallas_call(
        _tc_final_body,
        out_shape=jax.ShapeDtypeStruct((N_CLUSTERS, w2.shape[1]), jnp.float32),
    )(g, w1, b1.reshape(1, -1), w2, b2.reshape(1, -1))


def _pad_rows(a, n):
    return jnp.zeros((n, a.shape[1]), a.dtype).at[: a.shape[0]].set(a)


def kernel(x, pos, params, clusterID, edge_index):
    src = edge_index[0]
    dst = edge_index[1]

    (w1, b1), (w2, b2) = params["sa1_local"]
    (wg1, bg1) = params["sa1_global"][0]
    (u1, d1), (u2, d2) = params["sa2_local"]
    (wg2, bg2) = params["sa2_global"][0]
    (v1, c1), (v2, c2) = params["sa3"]
    (f1, e1), (f2, e2) = params["final"]

    w1x, w1p = w1[:128], jnp.zeros((16, 128), jnp.float32).at[:3].set(w1[128:])
    u1x, u1p = u1[:128], jnp.zeros((16, 256), jnp.float32).at[:3].set(u1[128:])
    v1x, v1p = v1[:256], jnp.zeros((16, 512), jnp.float32).at[:3].set(v1[256:])

    x_p = _pad_rows(x, N_PAD)
    pos16 = jnp.zeros((N_PAD, 16), jnp.float32).at[:N_NODES, :3].set(pos)
    cidf = jnp.full((N_PAD, 1), -1.0, jnp.float32).at[:N_NODES, 0].set(
        clusterID.astype(jnp.float32))

    bn = N_PAD // 4
    be = 3200

    mp = _sc_gather(pos16, dst, C=80, tc_tiling=False)

    a1 = _tc_prep1(x_p, pos16, w1x, w1p, b1, bn)
    ma1 = _sc_gather(a1, src, C=80)
    z1 = _tc_edge(ma1, mp, w1p, w2, b2, be)
    agg1 = _sc_segmax(z1, dst, K=128, CH=2560)

    a2 = _tc_prep2(agg1, pos16, wg1, bg1, u1x, u1p, d1, bn)
    ma2 = _sc_gather(a2, src, C=80)
    z2 = _tc_edge(ma2, mp, u1p, u2, d2, be)
    agg2 = _sc_segmax(z2, dst, K=128, CH=2560)

    g = _tc_tail(agg2, pos16, cidf, wg2, bg2, v1x, v1p, c1, v2, c2, bn)
    return _tc_final(g, f1, e1, f2, e2)


# speedup vs baseline: 1.6111x; 1.2197x over previous
import functools

import jax
import jax.numpy as jnp
from jax import lax
from jax.experimental import pallas as pl
from jax.experimental.pallas import tpu as pltpu
from jax.experimental.pallas import tpu_sc as plsc

N_NODES = 10000
N_EDGES = 320000
N_CLUSTERS = 16
NC, NS = 2, 16
NW = NC * NS
N_PAD = 10240
R_PER_W = N_PAD // NW


def _mesh():
    return plsc.VectorSubcoreMesh(
        core_axis_name="c", subcore_axis_name="s", num_cores=NC, num_subcores=NS)


def _wid():
    return lax.axis_index("s") * NC + lax.axis_index("c")


def _sc_gather(table, idx, *, C, tc_tiling=True):
    E = idx.shape[0]
    V, D = table.shape
    Ew = E // NW
    nchunks = Ew // C

    @functools.partial(
        pl.kernel,
        mesh=_mesh(),
        compiler_params=pltpu.CompilerParams(use_tc_tiling_on_sc=tc_tiling),
        out_type=jax.ShapeDtypeStruct((E, D), jnp.float32),
        scratch_types=[
            pltpu.VMEM((Ew,), jnp.int32),
            pltpu.VMEM((C, D), jnp.float32),
            pltpu.SemaphoreType.DMA,
        ],
    )
    def k(table_hbm, idx_hbm, out_hbm, ibuf, rbuf, sem):
        base = _wid() * Ew
        pltpu.sync_copy(idx_hbm.at[pl.ds(base, Ew)], ibuf)

        def chunk(i, _):
            off = i * C
            pltpu.async_copy(table_hbm.at[ibuf.at[pl.ds(off, C)]], rbuf, sem).wait()
            pltpu.sync_copy(rbuf, out_hbm.at[pl.ds(base + off, C)])
            return 0

        lax.fori_loop(0, nchunks, chunk, 0)

    return k(table, idx)


CAP = N_EDGES + 256
LW = 112


def _sc_segmax(z, dst, *, CH):
    E, D = z.shape
    K = 128
    nchunks = E // CH
    ngroups = CH // 16
    nsl = D // 16

    @functools.partial(
        pl.kernel,
        mesh=_mesh(),
        compiler_params=pltpu.CompilerParams(needs_layout_passes=False),
        out_type=(
            jax.ShapeDtypeStruct((N_PAD, D), jnp.float32),
            jax.ShapeDtypeStruct((NW * CAP,), jnp.int32),
            jax.ShapeDtypeStruct((NW * 16,), jnp.int32),
        ),
        scratch_types=[
            pltpu.VMEM((R_PER_W, D), jnp.float32),
            pltpu.VMEM((CH,), jnp.int32),
            pltpu.VMEM((K + 16,), jnp.int32),
            pltpu.VMEM((K + 16,), jnp.int32),
            pltpu.VMEM((K, D), jnp.float32),
            pltpu.VMEM((32,), jnp.int32),
            pltpu.VMEM((K,), jnp.int32),
            pltpu.SemaphoreType.DMA,
        ],
    )
    def k(z_hbm, dst_hbm, out_hbm, plist_hbm, cnt_hbm,
          tab, dbuf, eidb, ldb, rows, zbuf, pkb, sem):
        w = _wid()
        lo = w * R_PER_W
        hi = lo + R_PER_W
        zero16 = jnp.zeros((16,), jnp.int32)
        one16 = jnp.ones((16,), jnp.int32)
        iota16 = lax.iota(jnp.int32, 16)
        nine16 = jnp.full((16,), 9, jnp.int32)
        zero16f = jnp.zeros((16,), jnp.float32)
        lo16 = jnp.full((16,), lo, jnp.int32)
        hi16 = jnp.full((16,), hi, jnp.int32)
        lbase = w * CAP

        def init_tab(r, _):
            for s in range(nsl):
                tab[r, pl.ds(s * 16, 16)] = zero16f
            return 0

        lax.fori_loop(0, R_PER_W, init_tab, 0)

        for j in range(K // 16 + 1):
            eidb[pl.ds(j * 16, 16)] = iota16
            ldb[pl.ds(j * 16, 16)] = zero16
        zbuf[pl.ds(0, 16)] = zero16

        def maxin(n):
            pltpu.async_copy(z_hbm.at[eidb.at[pl.ds(0, K)]], rows, sem).wait()

            def per_edge(i, _):
                ld = ldb[pl.ds(i, 16)][0]
                for s in range(nsl):
                    sl = pl.ds(s * 16, 16)
                    tab[ld, sl] = jnp.maximum(tab[ld, sl], rows[i, sl])
                return 0

            lax.fori_loop(0, n, per_edge, 0)

        def pack_block():
            for j in range(K // 16):
                sl = pl.ds(j * 16, 16)
                pkb[sl] = lax.shift_left(eidb[sl], nine16) | ldb[sl]

        def chunk(ci, carry):
            pltpu.sync_copy(dst_hbm.at[pl.ds(ci * CH, CH)], dbuf)

            def group(g, carry):
                off, woff, eidv = carry
                d = dbuf[pl.ds(g * 16, 16)]
                mi = (jnp.where(d >= lo16, one16, zero16)
                      * jnp.where(d < hi16, one16, zero16))
                p = mi
                for sh in (1, 2, 4, 8):
                    zbuf[pl.ds(16, 16)] = p
                    p = p + zbuf[pl.ds(16 - sh, 16)]
                matched = jnp.where(mi > zero16,
                                    p + jnp.full((16,), off - 1, jnp.int32),
                                    jnp.full((16,), K, jnp.int32) + iota16)
                plsc.store_scatter(eidb, [matched], eidv)
                plsc.store_scatter(ldb, [matched], d - lo16)
                off = off + p[15]
                eidv = eidv + jnp.full((16,), 16, jnp.int32)

                def do_flush():
                    maxin(off)
                    pack_block()
                    pltpu.sync_copy(
                        pkb.at[pl.ds(0, LW)],
                        plist_hbm.at[pl.ds(
                            pl.multiple_of(lbase + woff, 8), LW)])
                    ev = eidb[pl.ds(LW, 16)]
                    lv = ldb[pl.ds(LW, 16)]
                    eidb[pl.ds(0, 16)] = ev
                    ldb[pl.ds(0, 16)] = lv
                    return (off - LW, woff + LW)

                off, woff = lax.cond(off >= LW, do_flush, lambda: (off, woff))
                return (off, woff, eidv)

            return lax.fori_loop(0, ngroups, group, carry)

        off, woff, _ = lax.fori_loop(
            0, nchunks, chunk, (jnp.int32(0), jnp.int32(0), iota16))
        maxin(off)
        pack_block()
        pltpu.sync_copy(
            pkb, plist_hbm.at[pl.ds(pl.multiple_of(lbase + woff, 8), K)])
        for j in range(K // 16):
            pkb[pl.ds(j * 16, 16)] = zero16
        pltpu.sync_copy(
            pkb, plist_hbm.at[pl.ds(pl.multiple_of(lbase + woff + K, 8), K)])
        pkb[pl.ds(0, 16)] = jnp.full((16,), woff + off, jnp.int32)
        pltpu.sync_copy(pkb.at[pl.ds(0, 16)], cnt_hbm.at[pl.ds(w * 16, 16)])
        pltpu.sync_copy(tab, out_hbm.at[pl.ds(lo, R_PER_W)])

    return k(z, dst)


def _sc_apply(z, plist, cnt):
    E, D = z.shape
    K = 128
    nsl = D // 16

    @functools.partial(
        pl.kernel,
        mesh=_mesh(),
        compiler_params=pltpu.CompilerParams(needs_layout_passes=False),
        out_type=jax.ShapeDtypeStruct((N_PAD, D), jnp.float32),
        scratch_types=[
            pltpu.VMEM((R_PER_W, D), jnp.float32),
            pltpu.VMEM((K + 16,), jnp.int32),
            pltpu.VMEM((K,), jnp.int32),
            pltpu.VMEM((K, D), jnp.float32),
            pltpu.VMEM((16,), jnp.int32),
            pltpu.SemaphoreType.DMA,
        ],
    )
    def k(z_hbm, plist_hbm, cnt_hbm, out_hbm, tab, pbuf, ibuf, rows, cbuf, sem):
        w = _wid()
        lo = w * R_PER_W
        lbase = w * CAP
        zero16f = jnp.zeros((16,), jnp.float32)
        nine16 = jnp.full((16,), 9, jnp.int32)

        def init_tab(r, _):
            for s in range(nsl):
                tab[r, pl.ds(s * 16, 16)] = zero16f
            return 0

        lax.fori_loop(0, R_PER_W, init_tab, 0)

        pltpu.sync_copy(cnt_hbm.at[pl.ds(w * 16, 16)], cbuf)
        n = cbuf[pl.ds(0, 16)][0]
        nblocks = (n + K - 1) // K

        def block(c, _):
            pltpu.sync_copy(plist_hbm.at[pl.ds(lbase + c * K, K)],
                            pbuf.at[pl.ds(0, K)])
            for j in range(K // 16):
                sl = pl.ds(j * 16, 16)
                ibuf[sl] = lax.shift_right_logical(pbuf[sl], nine16)
            pltpu.async_copy(z_hbm.at[ibuf], rows, sem).wait()
            m = jnp.minimum(jnp.int32(K), n - c * K)

            def per_edge(i, _):
                pv = pbuf[pl.ds(i, 16)][0]
                ld = pv & 511
                for s in range(nsl):
                    sl = pl.ds(s * 16, 16)
                    tab[ld, sl] = jnp.maximum(tab[ld, sl], rows[i, sl])
                return 0

            lax.fori_loop(0, m, per_edge, 0)
            return 0

        lax.fori_loop(0, nblocks, block, 0)
        pltpu.sync_copy(tab, out_hbm.at[pl.ds(lo, R_PER_W)])

    return k(z, plist, cnt)


def _node_block_spec(bn, d):
    return pl.BlockSpec((bn, d), lambda i: (i, 0))


def _full_spec(shape):
    return pl.BlockSpec(shape, lambda i: tuple(0 for _ in shape))


def _tc_prep1_body(x_ref, p_ref, wx_ref, wp_ref, b_ref, a_ref):
    a_ref[...] = (x_ref[...] @ wx_ref[...] + p_ref[...] @ wp_ref[...]
                  + b_ref[...])


def _tc_prep1(x_p, pos16, wx, wp16, b, bn):
    n = x_p.shape[0]
    dout = wx.shape[1]
    return pl.pallas_call(
        _tc_prep1_body,
        grid=(n // bn,),
        in_specs=[
            _node_block_spec(bn, x_p.shape[1]),
            _node_block_spec(bn, 16),
            _full_spec(wx.shape),
            _full_spec(wp16.shape),
            _full_spec((1, dout)),
        ],
        out_specs=_node_block_spec(bn, dout),
        out_shape=jax.ShapeDtypeStruct((n, dout), jnp.float32),
    )(x_p, pos16, wx, wp16, b.reshape(1, -1))


def _tc_edge_body(ma_ref, mp_ref, wp_ref, w2_ref, b2_ref, z_ref):
    m = jnp.maximum(ma_ref[...] - mp_ref[...] @ wp_ref[...], 0.0)
    z_ref[...] = m @ w2_ref[...] + b2_ref[...]


def _tc_edge(ma, mp, wp16, w2, b2, be):
    e, d = ma.shape
    dout = w2.shape[1]
    return pl.pallas_call(
        _tc_edge_body,
        grid=(e // be,),
        in_specs=[
            _node_block_spec(be, d),
            _node_block_spec(be, 16),
            _full_spec(wp16.shape),
            _full_spec(w2.shape),
            _full_spec((1, dout)),
        ],
        out_specs=_node_block_spec(be, dout),
        out_shape=jax.ShapeDtypeStruct((e, dout), jnp.float32),
    )(ma, mp, wp16, w2, b2.reshape(1, -1))


def _tc_prep2_body(agg_ref, p_ref, wg_ref, bg_ref, wx_ref, wp_ref, b_ref, a_ref):
    x1 = jnp.maximum(agg_ref[...] @ wg_ref[...] + bg_ref[...], 0.0)
    a_ref[...] = x1 @ wx_ref[...] + p_ref[...] @ wp_ref[...] + b_ref[...]


def _tc_prep2(agg, pos16, wg, bg, wx, wp16, b, bn):
    n = agg.shape[0]
    dout = wx.shape[1]
    return pl.pallas_call(
        _tc_prep2_body,
        grid=(n // bn,),
        in_specs=[
            _node_block_spec(bn, agg.shape[1]),
            _node_block_spec(bn, 16),
            _full_spec(wg.shape),
            _full_spec((1, wg.shape[1])),
            _full_spec(wx.shape),
            _full_spec(wp16.shape),
            _full_spec((1, dout)),
        ],
        out_specs=_node_block_spec(bn, dout),
        out_shape=jax.ShapeDtypeStruct((n, dout), jnp.float32),
    )(agg, pos16, wg, bg.reshape(1, -1), wx, wp16, b.reshape(1, -1))


def _tc_tail_body(agg_ref, p_ref, cid_ref, wg_ref, bg_ref, v1x_ref, v1p_ref,
                  c1_ref, v2_ref, c2_ref, g_ref):
    x2 = jnp.maximum(agg_ref[...] @ wg_ref[...] + bg_ref[...], 0.0)
    h1 = jnp.maximum(x2 @ v1x_ref[...] + p_ref[...] @ v1p_ref[...]
                     + c1_ref[...], 0.0)
    h = jnp.maximum(h1 @ v2_ref[...] + c2_ref[...], 0.0)

    @pl.when(pl.program_id(0) == 0)
    def _():
        g_ref[...] = jnp.zeros_like(g_ref)

    cid = cid_ref[...]
    for c in range(N_CLUSTERS):
        sel = jnp.where(cid == float(c), h, 0.0)
        g_ref[c, :] = jnp.maximum(g_ref[c, :], jnp.max(sel, axis=0))


def _tc_tail(agg, pos16, cidf, wg, bg, v1x, v1p16, c1, v2, c2, bn):
    n = agg.shape[0]
    dh = v2.shape[1]
    return pl.pallas_call(
        _tc_tail_body,
        grid=(n // bn,),
        in_specs=[
            _node_block_spec(bn, agg.shape[1]),
            _node_block_spec(bn, 16),
            _node_block_spec(bn, 1),
            _full_spec(wg.shape),
            _full_spec((1, wg.shape[1])),
            _full_spec(v1x.shape),
            _full_spec(v1p16.shape),
            _full_spec((1, dh)),
            _full_spec(v2.shape),
            _full_spec((1, dh)),
        ],
        out_specs=pl.BlockSpec((N_CLUSTERS, dh), lambda i: (0, 0)),
        out_shape=jax.ShapeDtypeStruct((N_CLUSTERS, dh), jnp.float32),
    )(agg, pos16, cidf, wg, bg.reshape(1, -1), v1x, v1p16, c1.reshape(1, -1),
      v2, c2.reshape(1, -1))


def _tc_final_body(g_ref, w1_ref, b1_ref, w2_ref, b2_ref, o_ref):
    h = jnp.maximum(g_ref[...] @ w1_ref[...] + b1_ref[...], 0.0)
    o_ref[...] = h @ w2_ref[...] + b2_ref[...]


def _tc_final(g, w1, b1, w2, b2):
    return pl.pallas_call(
        _tc_final_body,
        out_shape=jax.ShapeDtypeStruct((N_CLUSTERS, w2.shape[1]), jnp.float32),
    )(g, w1, b1.reshape(1, -1), w2, b2.reshape(1, -1))


def _pad_rows(a, n):
    return jnp.zeros((n, a.shape[1]), a.dtype).at[: a.shape[0]].set(a)


def kernel(x, pos, params, clusterID, edge_index):
    src = edge_index[0]
    dst = edge_index[1]

    (w1, b1), (w2, b2) = params["sa1_local"]
    (wg1, bg1) = params["sa1_global"][0]
    (u1, d1), (u2, d2) = params["sa2_local"]
    (wg2, bg2) = params["sa2_global"][0]
    (v1, c1), (v2, c2) = params["sa3"]
    (f1, e1), (f2, e2) = params["final"]

    w1x, w1p = w1[:128], jnp.zeros((16, 128), jnp.float32).at[:3].set(w1[128:])
    u1x, u1p = u1[:128], jnp.zeros((16, 256), jnp.float32).at[:3].set(u1[128:])
    v1x, v1p = v1[:256], jnp.zeros((16, 512), jnp.float32).at[:3].set(v1[256:])

    x_p = _pad_rows(x, N_PAD)
    pos16 = jnp.zeros((N_PAD, 16), jnp.float32).at[:N_NODES, :3].set(pos)
    cidf = jnp.full((N_PAD, 1), -1.0, jnp.float32).at[:N_NODES, 0].set(
        clusterID.astype(jnp.float32))

    bn = N_PAD // 4
    be = 3200

    mp = _sc_gather(pos16, dst, C=80, tc_tiling=False)

    a1 = _tc_prep1(x_p, pos16, w1x, w1p, b1, bn)
    ma1 = _sc_gather(a1, src, C=80)
    z1 = _tc_edge(ma1, mp, w1p, w2, b2, be)
    agg1, plist, cnt = _sc_segmax(z1, dst, CH=2560)

    a2 = _tc_prep2(agg1, pos16, wg1, bg1, u1x, u1p, d1, bn)
    ma2 = _sc_gather(a2, src, C=80)
    z2 = _tc_edge(ma2, mp, u1p, u2, d2, be)
    agg2 = _sc_apply(z2, plist, cnt)

    g = _tc_tail(agg2, pos16, cidf, wg2, bg2, v1x, v1p, c1, v2, c2, bn)
    return _tc_final(g, f1, e1, f2, e2)


# speedup vs baseline: 1.7050x; 1.0583x over previous
import functools

import jax
import jax.numpy as jnp
from jax import lax
from jax.experimental import pallas as pl
from jax.experimental.pallas import tpu as pltpu
from jax.experimental.pallas import tpu_sc as plsc

N_NODES = 10000
N_EDGES = 320000
N_CLUSTERS = 16
NC, NS = 2, 16
NW = NC * NS
N_PAD = 10240
R_PER_W = N_PAD // NW


def _mesh():
    return plsc.VectorSubcoreMesh(
        core_axis_name="c", subcore_axis_name="s", num_cores=NC, num_subcores=NS)


def _wid():
    return lax.axis_index("s") * NC + lax.axis_index("c")


def _sc_gather(table, idx, *, C, tc_tiling=True, U=5):
    E = idx.shape[0]
    V, D = table.shape
    Ew = E // NW
    nchunks = Ew // C
    assert nchunks % U == 0

    @functools.partial(
        pl.kernel,
        mesh=_mesh(),
        compiler_params=pltpu.CompilerParams(use_tc_tiling_on_sc=tc_tiling),
        out_type=jax.ShapeDtypeStruct((E, D), jnp.float32),
        scratch_types=[
            pltpu.VMEM((Ew,), jnp.int32),
            *[pltpu.VMEM((C, D), jnp.float32) for _ in range(U)],
            pltpu.SemaphoreType.DMA,
            pltpu.SemaphoreType.DMA,
        ],
    )
    def k(table_hbm, idx_hbm, out_hbm, ibuf, *rest):
        rbufs, (gsem, wsem) = rest[:U], rest[U:]
        base = _wid() * Ew
        pltpu.sync_copy(idx_hbm.at[pl.ds(base, Ew)], ibuf)

        def round_(p, _):
            gd = []
            for u in range(U):
                off = (p * U + u) * C
                gd.append(pltpu.async_copy(
                    table_hbm.at[ibuf.at[pl.ds(off, C)]], rbufs[u], gsem))
            wd = []
            for u in range(U):
                off = (p * U + u) * C
                gd[u].wait()
                wd.append(pltpu.async_copy(
                    rbufs[u], out_hbm.at[pl.ds(base + off, C)], wsem))
            for u in range(U):
                wd[u].wait()
            return 0

        lax.fori_loop(0, nchunks // U, round_, 0)

    return k(table, idx)


CAP = N_EDGES + 256
LW = 112


def _sc_segmax(z, dst, *, CH):
    E, D = z.shape
    K = 128
    nchunks = E // CH
    ngroups = CH // 16
    nsl = D // 16

    @functools.partial(
        pl.kernel,
        mesh=_mesh(),
        compiler_params=pltpu.CompilerParams(needs_layout_passes=False),
        out_type=(
            jax.ShapeDtypeStruct((N_PAD, D), jnp.float32),
            jax.ShapeDtypeStruct((NW * CAP,), jnp.int32),
            jax.ShapeDtypeStruct((NW * 16,), jnp.int32),
        ),
        scratch_types=[
            pltpu.VMEM((R_PER_W, D), jnp.float32),
            pltpu.VMEM((CH,), jnp.int32),
            pltpu.VMEM((K + 16,), jnp.int32),
            pltpu.VMEM((K + 16,), jnp.int32),
            pltpu.VMEM((K, D), jnp.float32),
            pltpu.VMEM((32,), jnp.int32),
            pltpu.VMEM((K,), jnp.int32),
            pltpu.SemaphoreType.DMA,
        ],
    )
    def k(z_hbm, dst_hbm, out_hbm, plist_hbm, cnt_hbm,
          tab, dbuf, eidb, ldb, rows, zbuf, pkb, sem):
        w = _wid()
        lo = w * R_PER_W
        hi = lo + R_PER_W
        zero16 = jnp.zeros((16,), jnp.int32)
        one16 = jnp.ones((16,), jnp.int32)
        iota16 = lax.iota(jnp.int32, 16)
        nine16 = jnp.full((16,), 9, jnp.int32)
        zero16f = jnp.zeros((16,), jnp.float32)
        lo16 = jnp.full((16,), lo, jnp.int32)
        hi16 = jnp.full((16,), hi, jnp.int32)
        lbase = w * CAP

        def init_tab(r, _):
            for s in range(nsl):
                tab[r, pl.ds(s * 16, 16)] = zero16f
            return 0

        lax.fori_loop(0, R_PER_W, init_tab, 0)

        for j in range(K // 16 + 1):
            eidb[pl.ds(j * 16, 16)] = iota16
            ldb[pl.ds(j * 16, 16)] = zero16
        zbuf[pl.ds(0, 16)] = zero16

        def maxin(n):
            pltpu.async_copy(z_hbm.at[eidb.at[pl.ds(0, K)]], rows, sem).wait()

            def per_edge(i, _):
                ld = ldb[pl.ds(i, 16)][0]
                for s in range(nsl):
                    sl = pl.ds(s * 16, 16)
                    tab[ld, sl] = jnp.maximum(tab[ld, sl], rows[i, sl])
                return 0

            lax.fori_loop(0, n, per_edge, 0)

        def pack_block():
            for j in range(K // 16):
                sl = pl.ds(j * 16, 16)
                pkb[sl] = lax.shift_left(eidb[sl], nine16) | ldb[sl]

        def chunk(ci, carry):
            pltpu.sync_copy(dst_hbm.at[pl.ds(ci * CH, CH)], dbuf)

            def group(g, carry):
                off, woff, eidv = carry
                d = dbuf[pl.ds(g * 16, 16)]
                mi = (jnp.where(d >= lo16, one16, zero16)
                      * jnp.where(d < hi16, one16, zero16))
                p = mi
                for sh in (1, 2, 4, 8):
                    zbuf[pl.ds(16, 16)] = p
                    p = p + zbuf[pl.ds(16 - sh, 16)]
                matched = jnp.where(mi > zero16,
                                    p + jnp.full((16,), off - 1, jnp.int32),
                                    jnp.full((16,), K, jnp.int32) + iota16)
                plsc.store_scatter(eidb, [matched], eidv)
                plsc.store_scatter(ldb, [matched], d - lo16)
                off = off + p[15]
                eidv = eidv + jnp.full((16,), 16, jnp.int32)

                def do_flush():
                    maxin(off)
                    pack_block()
                    pltpu.sync_copy(
                        pkb.at[pl.ds(0, LW)],
                        plist_hbm.at[pl.ds(
                            pl.multiple_of(lbase + woff, 8), LW)])
                    ev = eidb[pl.ds(LW, 16)]
                    lv = ldb[pl.ds(LW, 16)]
                    eidb[pl.ds(0, 16)] = ev
                    ldb[pl.ds(0, 16)] = lv
                    return (off - LW, woff + LW)

                off, woff = lax.cond(off >= LW, do_flush, lambda: (off, woff))
                return (off, woff, eidv)

            return lax.fori_loop(0, ngroups, group, carry)

        off, woff, _ = lax.fori_loop(
            0, nchunks, chunk, (jnp.int32(0), jnp.int32(0), iota16))
        maxin(off)
        pack_block()
        pltpu.sync_copy(
            pkb, plist_hbm.at[pl.ds(pl.multiple_of(lbase + woff, 8), K)])
        for j in range(K // 16):
            pkb[pl.ds(j * 16, 16)] = zero16
        pltpu.sync_copy(
            pkb, plist_hbm.at[pl.ds(pl.multiple_of(lbase + woff + K, 8), K)])
        pkb[pl.ds(0, 16)] = jnp.full((16,), woff + off, jnp.int32)
        pltpu.sync_copy(pkb.at[pl.ds(0, 16)], cnt_hbm.at[pl.ds(w * 16, 16)])
        pltpu.sync_copy(tab, out_hbm.at[pl.ds(lo, R_PER_W)])

    return k(z, dst)


def _sc_apply(z, plist, cnt):
    E, D = z.shape
    K = 128
    nsl = D // 16

    @functools.partial(
        pl.kernel,
        mesh=_mesh(),
        compiler_params=pltpu.CompilerParams(needs_layout_passes=False),
        out_type=jax.ShapeDtypeStruct((N_PAD, D), jnp.float32),
        scratch_types=[
            pltpu.VMEM((R_PER_W, D), jnp.float32),
            pltpu.VMEM((K + 16,), jnp.int32),
            pltpu.VMEM((K,), jnp.int32),
            pltpu.VMEM((K, D), jnp.float32),
            pltpu.VMEM((16,), jnp.int32),
            pltpu.SemaphoreType.DMA,
        ],
    )
    def k(z_hbm, plist_hbm, cnt_hbm, out_hbm, tab, pbuf, ibuf, rows, cbuf, sem):
        w = _wid()
        lo = w * R_PER_W
        lbase = w * CAP
        zero16f = jnp.zeros((16,), jnp.float32)
        nine16 = jnp.full((16,), 9, jnp.int32)

        def init_tab(r, _):
            for s in range(nsl):
                tab[r, pl.ds(s * 16, 16)] = zero16f
            return 0

        lax.fori_loop(0, R_PER_W, init_tab, 0)

        pltpu.sync_copy(cnt_hbm.at[pl.ds(w * 16, 16)], cbuf)
        n = cbuf[pl.ds(0, 16)][0]
        nblocks = (n + K - 1) // K

        def block(c, _):
            pltpu.sync_copy(plist_hbm.at[pl.ds(lbase + c * K, K)],
                            pbuf.at[pl.ds(0, K)])
            for j in range(K // 16):
                sl = pl.ds(j * 16, 16)
                ibuf[sl] = lax.shift_right_logical(pbuf[sl], nine16)
            pltpu.async_copy(z_hbm.at[ibuf], rows, sem).wait()
            m = jnp.minimum(jnp.int32(K), n - c * K)

            def per_edge(i, _):
                pv = pbuf[pl.ds(i, 16)][0]
                ld = pv & 511
                for s in range(nsl):
                    sl = pl.ds(s * 16, 16)
                    tab[ld, sl] = jnp.maximum(tab[ld, sl], rows[i, sl])
                return 0

            lax.fori_loop(0, m, per_edge, 0)
            return 0

        lax.fori_loop(0, nblocks, block, 0)
        pltpu.sync_copy(tab, out_hbm.at[pl.ds(lo, R_PER_W)])

    return k(z, plist, cnt)


def _node_block_spec(bn, d):
    return pl.BlockSpec((bn, d), lambda i: (i, 0))


def _full_spec(shape):
    return pl.BlockSpec(shape, lambda i: tuple(0 for _ in shape))


def _tc_prep1_body(x_ref, p_ref, wx_ref, wp_ref, b_ref, a_ref):
    a_ref[...] = (x_ref[...] @ wx_ref[...] + p_ref[...] @ wp_ref[...]
                  + b_ref[...])


def _tc_prep1(x_p, pos16, wx, wp16, b, bn):
    n = x_p.shape[0]
    dout = wx.shape[1]
    return pl.pallas_call(
        _tc_prep1_body,
        grid=(n // bn,),
        in_specs=[
            _node_block_spec(bn, x_p.shape[1]),
            _node_block_spec(bn, 16),
            _full_spec(wx.shape),
            _full_spec(wp16.shape),
            _full_spec((1, dout)),
        ],
        out_specs=_node_block_spec(bn, dout),
        out_shape=jax.ShapeDtypeStruct((n, dout), jnp.float32),
    )(x_p, pos16, wx, wp16, b.reshape(1, -1))


def _tc_edge_body(ma_ref, mp_ref, wp_ref, w2_ref, b2_ref, z_ref):
    m = jnp.maximum(ma_ref[...] - mp_ref[...] @ wp_ref[...], 0.0)
    z_ref[...] = m @ w2_ref[...] + b2_ref[...]


def _tc_edge(ma, mp, wp16, w2, b2, be):
    e, d = ma.shape
    dout = w2.shape[1]
    return pl.pallas_call(
        _tc_edge_body,
        grid=(e // be,),
        in_specs=[
            _node_block_spec(be, d),
            _node_block_spec(be, 16),
            _full_spec(wp16.shape),
            _full_spec(w2.shape),
            _full_spec((1, dout)),
        ],
        out_specs=_node_block_spec(be, dout),
        out_shape=jax.ShapeDtypeStruct((e, dout), jnp.float32),
    )(ma, mp, wp16, w2, b2.reshape(1, -1))


def _tc_prep2_body(agg_ref, p_ref, wg_ref, bg_ref, wx_ref, wp_ref, b_ref, a_ref):
    x1 = jnp.maximum(agg_ref[...] @ wg_ref[...] + bg_ref[...], 0.0)
    a_ref[...] = x1 @ wx_ref[...] + p_ref[...] @ wp_ref[...] + b_ref[...]


def _tc_prep2(agg, pos16, wg, bg, wx, wp16, b, bn):
    n = agg.shape[0]
    dout = wx.shape[1]
    return pl.pallas_call(
        _tc_prep2_body,
        grid=(n // bn,),
        in_specs=[
            _node_block_spec(bn, agg.shape[1]),
            _node_block_spec(bn, 16),
            _full_spec(wg.shape),
            _full_spec((1, wg.shape[1])),
            _full_spec(wx.shape),
            _full_spec(wp16.shape),
            _full_spec((1, dout)),
        ],
        out_specs=_node_block_spec(bn, dout),
        out_shape=jax.ShapeDtypeStruct((n, dout), jnp.float32),
    )(agg, pos16, wg, bg.reshape(1, -1), wx, wp16, b.reshape(1, -1))


def _tc_tail_body(agg_ref, p_ref, cid_ref, wg_ref, bg_ref, v1x_ref, v1p_ref,
                  c1_ref, v2_ref, c2_ref, g_ref):
    x2 = jnp.maximum(agg_ref[...] @ wg_ref[...] + bg_ref[...], 0.0)
    h1 = jnp.maximum(x2 @ v1x_ref[...] + p_ref[...] @ v1p_ref[...]
                     + c1_ref[...], 0.0)
    h = jnp.maximum(h1 @ v2_ref[...] + c2_ref[...], 0.0)

    @pl.when(pl.program_id(0) == 0)
    def _():
        g_ref[...] = jnp.zeros_like(g_ref)

    cid = cid_ref[...]
    for c in range(N_CLUSTERS):
        sel = jnp.where(cid == float(c), h, 0.0)
        g_ref[c, :] = jnp.maximum(g_ref[c, :], jnp.max(sel, axis=0))


def _tc_tail(agg, pos16, cidf, wg, bg, v1x, v1p16, c1, v2, c2, bn):
    n = agg.shape[0]
    dh = v2.shape[1]
    return pl.pallas_call(
        _tc_tail_body,
        grid=(n // bn,),
        in_specs=[
            _node_block_spec(bn, agg.shape[1]),
            _node_block_spec(bn, 16),
            _node_block_spec(bn, 1),
            _full_spec(wg.shape),
            _full_spec((1, wg.shape[1])),
            _full_spec(v1x.shape),
            _full_spec(v1p16.shape),
            _full_spec((1, dh)),
            _full_spec(v2.shape),
            _full_spec((1, dh)),
        ],
        out_specs=pl.BlockSpec((N_CLUSTERS, dh), lambda i: (0, 0)),
        out_shape=jax.ShapeDtypeStruct((N_CLUSTERS, dh), jnp.float32),
    )(agg, pos16, cidf, wg, bg.reshape(1, -1), v1x, v1p16, c1.reshape(1, -1),
      v2, c2.reshape(1, -1))


def _tc_final_body(g_ref, w1_ref, b1_ref, w2_ref, b2_ref, o_ref):
    h = jnp.maximum(g_ref[...] @ w1_ref[...] + b1_ref[...], 0.0)
    o_ref[...] = h @ w2_ref[...] + b2_ref[...]


def _tc_final(g, w1, b1, w2, b2):
    return pl.pallas_call(
        _tc_final_body,
        out_shape=jax.ShapeDtypeStruct((N_CLUSTERS, w2.shape[1]), jnp.float32),
    )(g, w1, b1.reshape(1, -1), w2, b2.reshape(1, -1))


def _pad_rows(a, n):
    return jnp.zeros((n, a.shape[1]), a.dtype).at[: a.shape[0]].set(a)


def kernel(x, pos, params, clusterID, edge_index):
    src = edge_index[0]
    dst = edge_index[1]

    (w1, b1), (w2, b2) = params["sa1_local"]
    (wg1, bg1) = params["sa1_global"][0]
    (u1, d1), (u2, d2) = params["sa2_local"]
    (wg2, bg2) = params["sa2_global"][0]
    (v1, c1), (v2, c2) = params["sa3"]
    (f1, e1), (f2, e2) = params["final"]

    w1x, w1p = w1[:128], jnp.zeros((16, 128), jnp.float32).at[:3].set(w1[128:])
    u1x, u1p = u1[:128], jnp.zeros((16, 256), jnp.float32).at[:3].set(u1[128:])
    v1x, v1p = v1[:256], jnp.zeros((16, 512), jnp.float32).at[:3].set(v1[256:])

    x_p = _pad_rows(x, N_PAD)
    pos16 = jnp.zeros((N_PAD, 16), jnp.float32).at[:N_NODES, :3].set(pos)
    cidf = jnp.full((N_PAD, 1), -1.0, jnp.float32).at[:N_NODES, 0].set(
        clusterID.astype(jnp.float32))

    bn = N_PAD // 4
    be = 3200

    mp = _sc_gather(pos16, dst, C=80, tc_tiling=False)

    a1 = _tc_prep1(x_p, pos16, w1x, w1p, b1, bn)
    ma1 = _sc_gather(a1, src, C=80)
    z1 = _tc_edge(ma1, mp, w1p, w2, b2, be)
    agg1, plist, cnt = _sc_segmax(z1, dst, CH=2560)

    a2 = _tc_prep2(agg1, pos16, wg1, bg1, u1x, u1p, d1, bn)
    ma2 = _sc_gather(a2, src, C=80)
    z2 = _tc_edge(ma2, mp, u1p, u2, d2, be)
    agg2 = _sc_apply(z2, plist, cnt)

    g = _tc_tail(agg2, pos16, cidf, wg2, bg2, v1x, v1p, c1, v2, c2, bn)
    return _tc_final(g, f1, e1, f2, e2)


# speedup vs baseline: 1.7314x; 1.0155x over previous
import functools

import jax
import jax.numpy as jnp
from jax import lax
from jax.experimental import pallas as pl
from jax.experimental.pallas import tpu as pltpu
from jax.experimental.pallas import tpu_sc as plsc

N_NODES = 10000
N_EDGES = 320000
N_CLUSTERS = 16
NC, NS = 2, 16
NW = NC * NS
N_PAD = 10240
R_PER_W = N_PAD // NW


def _mesh():
    return plsc.VectorSubcoreMesh(
        core_axis_name="c", subcore_axis_name="s", num_cores=NC, num_subcores=NS)


def _wid():
    return lax.axis_index("s") * NC + lax.axis_index("c")


def _sc_gather(table, idx, *, C, tc_tiling=True, U=5):
    E = idx.shape[0]
    V, D = table.shape
    Ew = E // NW
    nchunks = Ew // C
    assert nchunks % U == 0

    @functools.partial(
        pl.kernel,
        mesh=_mesh(),
        compiler_params=pltpu.CompilerParams(use_tc_tiling_on_sc=tc_tiling),
        out_type=jax.ShapeDtypeStruct((E, D), jnp.float32),
        scratch_types=[
            pltpu.VMEM((Ew,), jnp.int32),
            *[pltpu.VMEM((C, D), jnp.float32) for _ in range(U)],
            pltpu.SemaphoreType.DMA,
            pltpu.SemaphoreType.DMA,
        ],
    )
    def k(table_hbm, idx_hbm, out_hbm, ibuf, *rest):
        rbufs, (gsem, wsem) = rest[:U], rest[U:]
        base = _wid() * Ew
        pltpu.sync_copy(idx_hbm.at[pl.ds(base, Ew)], ibuf)

        def round_(p, _):
            gd = []
            for u in range(U):
                off = (p * U + u) * C
                gd.append(pltpu.async_copy(
                    table_hbm.at[ibuf.at[pl.ds(off, C)]], rbufs[u], gsem))
            wd = []
            for u in range(U):
                off = (p * U + u) * C
                gd[u].wait()
                wd.append(pltpu.async_copy(
                    rbufs[u], out_hbm.at[pl.ds(base + off, C)], wsem))
            for u in range(U):
                wd[u].wait()
            return 0

        lax.fori_loop(0, nchunks // U, round_, 0)

    return k(table, idx)


CAP = N_EDGES + 256
LW = 112


def _sc_segmax(z, dst, *, CH):
    E, D = z.shape
    K = 128
    nchunks = E // CH
    ngroups = CH // 16
    nsl = D // 16

    @functools.partial(
        pl.kernel,
        mesh=_mesh(),
        compiler_params=pltpu.CompilerParams(needs_layout_passes=False),
        out_type=(
            jax.ShapeDtypeStruct((N_PAD, D), jnp.float32),
            jax.ShapeDtypeStruct((NW * CAP,), jnp.int32),
            jax.ShapeDtypeStruct((NW * 16,), jnp.int32),
        ),
        scratch_types=[
            pltpu.VMEM((R_PER_W, D), jnp.float32),
            pltpu.VMEM((CH,), jnp.int32),
            pltpu.VMEM((K + 16,), jnp.int32),
            pltpu.VMEM((K + 16,), jnp.int32),
            pltpu.VMEM((K, D), jnp.float32),
            pltpu.VMEM((32,), jnp.int32),
            pltpu.VMEM((K,), jnp.int32),
            pltpu.SemaphoreType.DMA,
        ],
    )
    def k(z_hbm, dst_hbm, out_hbm, plist_hbm, cnt_hbm,
          tab, dbuf, eidb, ldb, rows, zbuf, pkb, sem):
        w = _wid()
        lo = w * R_PER_W
        hi = lo + R_PER_W
        zero16 = jnp.zeros((16,), jnp.int32)
        one16 = jnp.ones((16,), jnp.int32)
        iota16 = lax.iota(jnp.int32, 16)
        nine16 = jnp.full((16,), 9, jnp.int32)
        zero16f = jnp.zeros((16,), jnp.float32)
        lo16 = jnp.full((16,), lo, jnp.int32)
        hi16 = jnp.full((16,), hi, jnp.int32)
        lbase = w * CAP

        def init_tab(r, _):
            for s in range(nsl):
                tab[r, pl.ds(s * 16, 16)] = zero16f
            return 0

        lax.fori_loop(0, R_PER_W, init_tab, 0)

        for j in range(K // 16 + 1):
            eidb[pl.ds(j * 16, 16)] = iota16
            ldb[pl.ds(j * 16, 16)] = zero16
        zbuf[pl.ds(0, 16)] = zero16

        def maxin(n):
            pltpu.async_copy(z_hbm.at[eidb.at[pl.ds(0, K)]], rows, sem).wait()

            def per_edge(i, _):
                ld = ldb[pl.ds(i, 16)][0]
                for s in range(nsl):
                    sl = pl.ds(s * 16, 16)
                    tab[ld, sl] = jnp.maximum(tab[ld, sl], rows[i, sl])
                return 0

            lax.fori_loop(0, n, per_edge, 0)

        def pack_block():
            for j in range(K // 16):
                sl = pl.ds(j * 16, 16)
                pkb[sl] = lax.shift_left(eidb[sl], nine16) | ldb[sl]

        def chunk(ci, carry):
            pltpu.sync_copy(dst_hbm.at[pl.ds(ci * CH, CH)], dbuf)

            def group(g, carry):
                off, woff, eidv = carry
                d = dbuf[pl.ds(g * 16, 16)]
                mi = (jnp.where(d >= lo16, one16, zero16)
                      * jnp.where(d < hi16, one16, zero16))
                p = mi
                for sh in (1, 2, 4, 8):
                    zbuf[pl.ds(16, 16)] = p
                    p = p + zbuf[pl.ds(16 - sh, 16)]
                matched = jnp.where(mi > zero16,
                                    p + jnp.full((16,), off - 1, jnp.int32),
                                    jnp.full((16,), K, jnp.int32) + iota16)
                plsc.store_scatter(eidb, [matched], eidv)
                plsc.store_scatter(ldb, [matched], d - lo16)
                off = off + p[15]
                eidv = eidv + jnp.full((16,), 16, jnp.int32)

                def do_flush():
                    maxin(off)
                    pack_block()
                    pltpu.sync_copy(
                        pkb.at[pl.ds(0, LW)],
                        plist_hbm.at[pl.ds(
                            pl.multiple_of(lbase + woff, 8), LW)])
                    ev = eidb[pl.ds(LW, 16)]
                    lv = ldb[pl.ds(LW, 16)]
                    eidb[pl.ds(0, 16)] = ev
                    ldb[pl.ds(0, 16)] = lv
                    return (off - LW, woff + LW)

                off, woff = lax.cond(off >= LW, do_flush, lambda: (off, woff))
                return (off, woff, eidv)

            return lax.fori_loop(0, ngroups, group, carry)

        off, woff, _ = lax.fori_loop(
            0, nchunks, chunk, (jnp.int32(0), jnp.int32(0), iota16))
        maxin(off)
        pack_block()
        pltpu.sync_copy(
            pkb, plist_hbm.at[pl.ds(pl.multiple_of(lbase + woff, 8), K)])
        for j in range(K // 16):
            pkb[pl.ds(j * 16, 16)] = zero16
        pltpu.sync_copy(
            pkb, plist_hbm.at[pl.ds(pl.multiple_of(lbase + woff + K, 8), K)])
        pltpu.sync_copy(
            pkb,
            plist_hbm.at[pl.ds(pl.multiple_of(lbase + woff + 2 * K, 8), K)])
        pkb[pl.ds(0, 16)] = jnp.full((16,), woff + off, jnp.int32)
        pltpu.sync_copy(pkb.at[pl.ds(0, 16)], cnt_hbm.at[pl.ds(w * 16, 16)])
        pltpu.sync_copy(tab, out_hbm.at[pl.ds(lo, R_PER_W)])

    return k(z, dst)


def _sc_apply(z, plist, cnt):
    E, D = z.shape
    KB = 64 if D > 128 else 128
    nsl = D // 16

    @functools.partial(
        pl.kernel,
        mesh=_mesh(),
        compiler_params=pltpu.CompilerParams(needs_layout_passes=False),
        out_type=jax.ShapeDtypeStruct((N_PAD, D), jnp.float32),
        scratch_types=[
            pltpu.VMEM((R_PER_W, D), jnp.float32),
            pltpu.VMEM((KB + 16,), jnp.int32),
            pltpu.VMEM((KB + 16,), jnp.int32),
            pltpu.VMEM((KB,), jnp.int32),
            pltpu.VMEM((KB,), jnp.int32),
            pltpu.VMEM((KB, D), jnp.float32),
            pltpu.VMEM((KB, D), jnp.float32),
            pltpu.VMEM((16,), jnp.int32),
            pltpu.SemaphoreType.DMA,
        ],
    )
    def k(z_hbm, plist_hbm, cnt_hbm, out_hbm, tab, pb0, pb1, ib0, ib1,
          rw0, rw1, cbuf, sem):
        w = _wid()
        lo = w * R_PER_W
        lbase = w * CAP
        zero16f = jnp.zeros((16,), jnp.float32)
        nine16 = jnp.full((16,), 9, jnp.int32)
        pbufs, ibufs, rws = (pb0, pb1), (ib0, ib1), (rw0, rw1)

        def init_tab(r, _):
            for s in range(nsl):
                tab[r, pl.ds(s * 16, 16)] = zero16f
            return 0

        lax.fori_loop(0, R_PER_W, init_tab, 0)

        pltpu.sync_copy(cnt_hbm.at[pl.ds(w * 16, 16)], cbuf)
        n = cbuf[pl.ds(0, 16)][0]
        npairs = (n + 2 * KB - 1) // (2 * KB)

        def pair(q, _):
            gd = []
            for u in range(2):
                c = 2 * q + u
                pltpu.sync_copy(
                    plist_hbm.at[pl.ds(
                        pl.multiple_of(lbase + c * KB, 8), KB)],
                    pbufs[u].at[pl.ds(0, KB)])
                for j in range(KB // 16):
                    sl = pl.ds(j * 16, 16)
                    ibufs[u][sl] = lax.shift_right_logical(
                        pbufs[u][sl], nine16)
                gd.append(pltpu.async_copy(z_hbm.at[ibufs[u]], rws[u], sem))
            for u in range(2):
                c = 2 * q + u
                gd[u].wait()
                m = jnp.maximum(
                    jnp.int32(0), jnp.minimum(jnp.int32(KB), n - c * KB))

                def per_edge(i, _, u=u):
                    pv = pbufs[u][pl.ds(i, 16)][0]
                    ld = pv & 511
                    for s in range(nsl):
                        sl = pl.ds(s * 16, 16)
                        tab[ld, sl] = jnp.maximum(tab[ld, sl],
                                                  rws[u][i, sl])
                    return 0

                lax.fori_loop(0, m, per_edge, 0)
            return 0

        lax.fori_loop(0, npairs, pair, 0)
        pltpu.sync_copy(tab, out_hbm.at[pl.ds(lo, R_PER_W)])

    return k(z, plist, cnt)


def _node_block_spec(bn, d):
    return pl.BlockSpec((bn, d), lambda i: (i, 0))


def _full_spec(shape):
    return pl.BlockSpec(shape, lambda i: tuple(0 for _ in shape))


def _tc_prep1_body(x_ref, p_ref, wx_ref, wp_ref, b_ref, a_ref):
    a_ref[...] = (x_ref[...] @ wx_ref[...] + p_ref[...] @ wp_ref[...]
                  + b_ref[...])


def _tc_prep1(x_p, pos16, wx, wp16, b, bn):
    n = x_p.shape[0]
    dout = wx.shape[1]
    return pl.pallas_call(
        _tc_prep1_body,
        grid=(n // bn,),
        in_specs=[
            _node_block_spec(bn, x_p.shape[1]),
            _node_block_spec(bn, 16),
            _full_spec(wx.shape),
            _full_spec(wp16.shape),
            _full_spec((1, dout)),
        ],
        out_specs=_node_block_spec(bn, dout),
        out_shape=jax.ShapeDtypeStruct((n, dout), jnp.float32),
    )(x_p, pos16, wx, wp16, b.reshape(1, -1))


def _tc_edge_body(ma_ref, mp_ref, wp_ref, w2_ref, b2_ref, z_ref):
    m = jnp.maximum(ma_ref[...] - mp_ref[...] @ wp_ref[...], 0.0)
    z_ref[...] = m @ w2_ref[...] + b2_ref[...]


def _tc_edge(ma, mp, wp16, w2, b2, be):
    e, d = ma.shape
    dout = w2.shape[1]
    return pl.pallas_call(
        _tc_edge_body,
        grid=(e // be,),
        in_specs=[
            _node_block_spec(be, d),
            _node_block_spec(be, 16),
            _full_spec(wp16.shape),
            _full_spec(w2.shape),
            _full_spec((1, dout)),
        ],
        out_specs=_node_block_spec(be, dout),
        out_shape=jax.ShapeDtypeStruct((e, dout), jnp.float32),
    )(ma, mp, wp16, w2, b2.reshape(1, -1))


def _tc_prep2_body(agg_ref, p_ref, wg_ref, bg_ref, wx_ref, wp_ref, b_ref, a_ref):
    x1 = jnp.maximum(agg_ref[...] @ wg_ref[...] + bg_ref[...], 0.0)
    a_ref[...] = x1 @ wx_ref[...] + p_ref[...] @ wp_ref[...] + b_ref[...]


def _tc_prep2(agg, pos16, wg, bg, wx, wp16, b, bn):
    n = agg.shape[0]
    dout = wx.shape[1]
    return pl.pallas_call(
        _tc_prep2_body,
        grid=(n // bn,),
        in_specs=[
            _node_block_spec(bn, agg.shape[1]),
            _node_block_spec(bn, 16),
            _full_spec(wg.shape),
            _full_spec((1, wg.shape[1])),
            _full_spec(wx.shape),
            _full_spec(wp16.shape),
            _full_spec((1, dout)),
        ],
        out_specs=_node_block_spec(bn, dout),
        out_shape=jax.ShapeDtypeStruct((n, dout), jnp.float32),
    )(agg, pos16, wg, bg.reshape(1, -1), wx, wp16, b.reshape(1, -1))


def _tc_tail_body(agg_ref, p_ref, cid_ref, wg_ref, bg_ref, v1x_ref, v1p_ref,
                  c1_ref, v2_ref, c2_ref, g_ref):
    x2 = jnp.maximum(agg_ref[...] @ wg_ref[...] + bg_ref[...], 0.0)
    h1 = jnp.maximum(x2 @ v1x_ref[...] + p_ref[...] @ v1p_ref[...]
                     + c1_ref[...], 0.0)
    h = jnp.maximum(h1 @ v2_ref[...] + c2_ref[...], 0.0)

    @pl.when(pl.program_id(0) == 0)
    def _():
        g_ref[...] = jnp.zeros_like(g_ref)

    cid = cid_ref[...]
    for c in range(N_CLUSTERS):
        sel = jnp.where(cid == float(c), h, 0.0)
        g_ref[c, :] = jnp.maximum(g_ref[c, :], jnp.max(sel, axis=0))


def _tc_tail(agg, pos16, cidf, wg, bg, v1x, v1p16, c1, v2, c2, bn):
    n = agg.shape[0]
    dh = v2.shape[1]
    return pl.pallas_call(
        _tc_tail_body,
        grid=(n // bn,),
        in_specs=[
            _node_block_spec(bn, agg.shape[1]),
            _node_block_spec(bn, 16),
            _node_block_spec(bn, 1),
            _full_spec(wg.shape),
            _full_spec((1, wg.shape[1])),
            _full_spec(v1x.shape),
            _full_spec(v1p16.shape),
            _full_spec((1, dh)),
            _full_spec(v2.shape),
            _full_spec((1, dh)),
        ],
        out_specs=pl.BlockSpec((N_CLUSTERS, dh), lambda i: (0, 0)),
        out_shape=jax.ShapeDtypeStruct((N_CLUSTERS, dh), jnp.float32),
    )(agg, pos16, cidf, wg, bg.reshape(1, -1), v1x, v1p16, c1.reshape(1, -1),
      v2, c2.reshape(1, -1))


def _tc_final_body(g_ref, w1_ref, b1_ref, w2_ref, b2_ref, o_ref):
    h = jnp.maximum(g_ref[...] @ w1_ref[...] + b1_ref[...], 0.0)
    o_ref[...] = h @ w2_ref[...] + b2_ref[...]


def _tc_final(g, w1, b1, w2, b2):
    return pl.pallas_call(
        _tc_final_body,
        out_shape=jax.ShapeDtypeStruct((N_CLUSTERS, w2.shape[1]), jnp.float32),
    )(g, w1, b1.reshape(1, -1), w2, b2.reshape(1, -1))


def _pad_rows(a, n):
    return jnp.zeros((n, a.shape[1]), a.dtype).at[: a.shape[0]].set(a)


def kernel(x, pos, params, clusterID, edge_index):
    src = edge_index[0]
    dst = edge_index[1]

    (w1, b1), (w2, b2) = params["sa1_local"]
    (wg1, bg1) = params["sa1_global"][0]
    (u1, d1), (u2, d2) = params["sa2_local"]
    (wg2, bg2) = params["sa2_global"][0]
    (v1, c1), (v2, c2) = params["sa3"]
    (f1, e1), (f2, e2) = params["final"]

    w1x, w1p = w1[:128], jnp.zeros((16, 128), jnp.float32).at[:3].set(w1[128:])
    u1x, u1p = u1[:128], jnp.zeros((16, 256), jnp.float32).at[:3].set(u1[128:])
    v1x, v1p = v1[:256], jnp.zeros((16, 512), jnp.float32).at[:3].set(v1[256:])

    x_p = _pad_rows(x, N_PAD)
    pos16 = jnp.zeros((N_PAD, 16), jnp.float32).at[:N_NODES, :3].set(pos)
    cidf = jnp.full((N_PAD, 1), -1.0, jnp.float32).at[:N_NODES, 0].set(
        clusterID.astype(jnp.float32))

    bn = N_PAD // 4
    be = 3200

    mp = _sc_gather(pos16, dst, C=80, tc_tiling=False)

    a1 = _tc_prep1(x_p, pos16, w1x, w1p, b1, bn)
    ma1 = _sc_gather(a1, src, C=80)
    z1 = _tc_edge(ma1, mp, w1p, w2, b2, be)
    agg1, plist, cnt = _sc_segmax(z1, dst, CH=2560)

    a2 = _tc_prep2(agg1, pos16, wg1, bg1, u1x, u1p, d1, bn)
    ma2 = _sc_gather(a2, src, C=80)
    z2 = _tc_edge(ma2, mp, u1p, u2, d2, be)
    agg2 = _sc_apply(z2, plist, cnt)

    g = _tc_tail(agg2, pos16, cidf, wg2, bg2, v1x, v1p, c1, v2, c2, bn)
    return _tc_final(g, f1, e1, f2, e2)
